# baseline (speedup 1.0000x reference)
import functools

import jax
import jax.numpy as jnp
from jax import lax
from jax.experimental import pallas as pl
from jax.experimental.pallas import tpu as pltpu
from jax.experimental.pallas import tpu_sc as plsc

_B = 4096
_C = 1000
_NC = 2
_NS = 16
_L = 16
_NW = _NC * _NS
_ROWS_PER_W = _B // _NW
_GROUPS = _ROWS_PER_W // _L

_R = 1.0
_DELTA = 0.5
_UNROLL = 8

_mesh = plsc.VectorSubcoreMesh(
    core_axis_name="c", subcore_axis_name="s",
    num_cores=_NC, num_subcores=_NS)


@functools.partial(
    pl.kernel,
    out_type=jax.ShapeDtypeStruct((_NW * _L,), jnp.float32),
    mesh=_mesh,
    scratch_types=[
        pltpu.VMEM((_L, _C), jnp.float32),
        pltpu.VMEM((_L, _C), jnp.float32),
        pltpu.VMEM((_ROWS_PER_W,), jnp.int32),
        pltpu.VMEM((_L,), jnp.float32),
        pltpu.SemaphoreType.DMA,
        pltpu.SemaphoreType.DMA,
    ],
    compiler_params=pltpu.CompilerParams(needs_layout_passes=False),
)
def _npt_loss_sc(dot_hbm, tgt_hbm, out_hbm, buf0, buf1, tgt_v, out_v,
                 sem0, sem1):
    wid = lax.axis_index("s") * _NC + lax.axis_index("c")
    base = wid * _ROWS_PER_W
    pltpu.sync_copy(tgt_hbm.at[pl.ds(base, _ROWS_PER_W)], tgt_v)

    bufs = (buf0, buf1)
    sems = (sem0, sem1)
    copies = [pltpu.async_copy(
        dot_hbm.at[pl.ds(base, _L)], buf0, sem0), None]

    row_iota = lax.iota(jnp.int32, _L)
    zeros = jnp.zeros((_L,), jnp.float32)
    neg_inf = jnp.full((_L,), -jnp.inf, jnp.float32)
    acc = zeros

    for g in range(_GROUPS):
        buf = bufs[g % 2]
        copies[g % 2].wait()
        if g + 1 < _GROUPS:
            copies[(g + 1) % 2] = pltpu.async_copy(
                dot_hbm.at[pl.ds(base + (g + 1) * _L, _L)],
                bufs[(g + 1) % 2], sems[(g + 1) % 2])

        tgt = tgt_v[pl.ds(g * _L, _L)]
        tvec = plsc.load_gather(buf, [row_iota, tgt])
        plsc.store_scatter(buf, [row_iota, tgt], zeros)

        def body(blk, carry):
            idx, m1a, m2a, m1b, m2b = carry
            for k in range(0, _UNROLL, 2):
                xa = plsc.load_gather(buf, [row_iota, idx])
                xb = plsc.load_gather(buf, [row_iota, idx + 1])
                idx = idx + 2
                m2a = jnp.maximum(m2a, jnp.minimum(m1a, xa))
                m1a = jnp.maximum(m1a, xa)
                m2b = jnp.maximum(m2b, jnp.minimum(m1b, xb))
                m1b = jnp.maximum(m1b, xb)
            return (idx, m1a, m2a, m1b, m2b)

        _, m1a, m2a, m1b, m2b = lax.fori_loop(
            0, _C // _UNROLL, body,
            (jnp.zeros((_L,), jnp.int32), neg_inf, neg_inf, neg_inf, neg_inf))
        m1 = jnp.maximum(m1a, m1b)
        m2 = jnp.maximum(jnp.minimum(m1a, m1b), jnp.maximum(m2a, m2b))

        l1 = jnp.maximum(m1 - tvec + _DELTA, 0.0)
        l2 = jnp.maximum(m2 - tvec + _DELTA, 0.0)
        acc = acc + (l1 + l2) * (2.0 * _R)

    out_v[...] = acc
    pltpu.sync_copy(out_v, out_hbm.at[pl.ds(wid * _L, _L)])


def kernel(dot_p, target):
    partials = _npt_loss_sc(dot_p, target.astype(jnp.int32))
    return jnp.sum(partials) / _B


# speedup vs baseline: 1.7937x; 1.7937x over previous
import functools

import jax
import jax.numpy as jnp
from jax import lax
from jax.experimental import pallas as pl
from jax.experimental.pallas import tpu as pltpu
from jax.experimental.pallas import tpu_sc as plsc

_B = 4096
_C = 1000
_NC = 2
_NS = 16
_L = 16
_NW = _NC * _NS
_ROWS_PER_W = _B // _NW
_GROUPS = _ROWS_PER_W // _L

_R = 1.0
_DELTA = 0.5

_FULL_CHUNKS = [16 * k for k in range(62)]
_TAIL_OFF = 984

_STRIDE = _L + 1

_mesh = plsc.VectorSubcoreMesh(
    core_axis_name="c", subcore_axis_name="s",
    num_cores=_NC, num_subcores=_NS)


@functools.partial(
    pl.kernel,
    out_type=jax.ShapeDtypeStruct((_NW * _L,), jnp.float32),
    mesh=_mesh,
    scratch_types=[
        pltpu.VMEM((_L, _C), jnp.float32),
        pltpu.VMEM((_L, _C), jnp.float32),
        pltpu.VMEM((_ROWS_PER_W,), jnp.int32),
        pltpu.VMEM((2 * _L * _STRIDE,), jnp.float32),
        pltpu.VMEM((_L,), jnp.float32),
        pltpu.SemaphoreType.DMA,
        pltpu.SemaphoreType.DMA,
    ],
    compiler_params=pltpu.CompilerParams(needs_layout_passes=False),
)
def _npt_loss_sc(dot_hbm, tgt_hbm, out_hbm, buf0, buf1, tgt_v, stats_v,
                 out_v, sem0, sem1):
    wid = lax.axis_index("s") * _NC + lax.axis_index("c")
    base = wid * _ROWS_PER_W
    pltpu.sync_copy(tgt_hbm.at[pl.ds(base, _ROWS_PER_W)], tgt_v)

    bufs = (buf0, buf1)
    sems = (sem0, sem1)
    copies = [pltpu.async_copy(
        dot_hbm.at[pl.ds(base, _L)], buf0, sem0), None]

    row_iota = lax.iota(jnp.int32, _L)
    lane_lt8 = row_iota < 8
    zeros = jnp.zeros((_L,), jnp.float32)
    neg_inf = jnp.full((_L,), -jnp.inf, jnp.float32)
    acc = zeros

    def merge2(m1p, m2p, m1q, m2q):
        return (jnp.maximum(m1p, m1q),
                jnp.maximum(jnp.minimum(m1p, m1q),
                            jnp.maximum(m2p, m2q)))

    for g in range(_GROUPS):
        buf = bufs[g % 2]
        copies[g % 2].wait()
        if g + 1 < _GROUPS:
            copies[(g + 1) % 2] = pltpu.async_copy(
                dot_hbm.at[pl.ds(base + (g + 1) * _L, _L)],
                bufs[(g + 1) % 2], sems[(g + 1) % 2])

        tgt = tgt_v[pl.ds(g * _L, _L)]
        tvec = plsc.load_gather(buf, [row_iota, tgt])
        plsc.store_scatter(buf, [row_iota, tgt], zeros)

        def row_body(r, carry):
            m1 = [neg_inf] * 4
            m2 = [neg_inf] * 4
            for i, off in enumerate(_FULL_CHUNKS):
                x = buf[r, pl.ds(off, _L)]
                p = i % 4
                m2[p] = jnp.maximum(m2[p], jnp.minimum(m1[p], x))
                m1[p] = jnp.maximum(m1[p], x)
            xt = buf[r, pl.ds(_TAIL_OFF, _L)]
            xt = jnp.where(lane_lt8, neg_inf, xt)
            m2[2] = jnp.maximum(m2[2], jnp.minimum(m1[2], xt))
            m1[2] = jnp.maximum(m1[2], xt)
            a1, a2 = merge2(m1[0], m2[0], m1[1], m2[1])
            b1, b2 = merge2(m1[2], m2[2], m1[3], m2[3])
            r1, r2 = merge2(a1, a2, b1, b2)
            sbase = r * _STRIDE
            plsc.store_scatter(stats_v, [row_iota + sbase], r1)
            plsc.store_scatter(stats_v, [row_iota + (sbase + _L * _STRIDE)],
                               r2)
            return carry

        lax.fori_loop(0, _L, row_body, 0)

        iota_s = row_iota * _STRIDE
        m1g = neg_inf
        m2g = neg_inf
        for j in range(_L):
            va = plsc.load_gather(stats_v, [iota_s + j])
            vb = plsc.load_gather(stats_v, [iota_s + (_L * _STRIDE + j)])
            m2g = jnp.maximum(m2g, jnp.minimum(m1g, va))
            m1g = jnp.maximum(m1g, va)
            m2g = jnp.maximum(m2g, jnp.minimum(m1g, vb))
            m1g = jnp.maximum(m1g, vb)

        l1 = jnp.maximum(m1g - tvec + _DELTA, 0.0)
        l2 = jnp.maximum(m2g - tvec + _DELTA, 0.0)
        acc = acc + (l1 + l2) * (2.0 * _R)

    out_v[...] = acc
    pltpu.sync_copy(out_v, out_hbm.at[pl.ds(wid * _L, _L)])


def kernel(dot_p, target):
    partials = _npt_loss_sc(dot_p, target.astype(jnp.int32))
    return jnp.sum(partials) / _B


# speedup vs baseline: 2.6574x; 1.4816x over previous
import functools

import jax
import jax.numpy as jnp
from jax import lax
from jax.experimental import pallas as pl
from jax.experimental.pallas import tpu as pltpu
from jax.experimental.pallas import tpu_sc as plsc

_B = 4096
_C = 1000
_NC = 2
_NS = 16
_L = 16
_NW = _NC * _NS
_ROWS_PER_W = _B // _NW
_NG = _ROWS_PER_W // _L

_CT = 200
_NCHUNK = _C // _CT

_R = 1.0
_DELTA = 0.5

_mesh = plsc.VectorSubcoreMesh(
    core_axis_name="c", subcore_axis_name="s",
    num_cores=_NC, num_subcores=_NS)


@functools.partial(
    pl.kernel,
    out_type=jax.ShapeDtypeStruct((_NW * _L,), jnp.float32),
    mesh=_mesh,
    scratch_types=[
        pltpu.VMEM((_CT, _ROWS_PER_W), jnp.float32),
        pltpu.VMEM((_CT, _ROWS_PER_W), jnp.float32),
        pltpu.VMEM((_ROWS_PER_W,), jnp.int32),
        pltpu.VMEM((_L,), jnp.float32),
        pltpu.SemaphoreType.DMA,
        pltpu.SemaphoreType.DMA,
    ],
    compiler_params=pltpu.CompilerParams(needs_layout_passes=False),
)
def _npt_loss_sc(dotT_hbm, tgt_hbm, out_hbm, buf0, buf1, tgt_v, out_v,
                 sem0, sem1):
    wid = lax.axis_index("s") * _NC + lax.axis_index("c")
    rbase = pl.multiple_of(wid * _ROWS_PER_W, _ROWS_PER_W)
    pltpu.sync_copy(tgt_hbm.at[pl.ds(rbase, _ROWS_PER_W)], tgt_v)

    bufs = (buf0, buf1)
    sems = (sem0, sem1)
    copies = [pltpu.async_copy(
        dotT_hbm.at[pl.ds(0, _CT), pl.ds(rbase, _ROWS_PER_W)], buf0, sem0),
        None]

    row_iota = lax.iota(jnp.int32, _L)
    zeros = jnp.zeros((_L,), jnp.float32)
    neg_inf = jnp.full((_L,), -jnp.inf, jnp.float32)

    tgts = [tgt_v[pl.ds(16 * l, _L)] for l in range(_NG)]
    cols = [row_iota + 16 * l for l in range(_NG)]
    m1 = [neg_inf] * _NG
    m2 = [neg_inf] * _NG
    tv = [zeros] * _NG

    for ch in range(_NCHUNK):
        buf = bufs[ch % 2]
        copies[ch % 2].wait()
        if ch + 1 < _NCHUNK:
            copies[(ch + 1) % 2] = pltpu.async_copy(
                dotT_hbm.at[pl.ds((ch + 1) * _CT, _CT),
                            pl.ds(rbase, _ROWS_PER_W)],
                bufs[(ch + 1) % 2], sems[(ch + 1) % 2])

        c0 = ch * _CT
        for l in range(_NG):
            inr = (tgts[l] >= c0) & (tgts[l] < c0 + _CT)
            idx = jnp.clip(tgts[l] - c0, 0, _CT - 1)
            got = plsc.load_gather(buf, [idx, cols[l]], mask=inr)
            tv[l] = jnp.where(inr, got, tv[l])
            plsc.store_scatter(buf, [idx, cols[l]], zeros, mask=inr)

        def body(rr, carry):
            cm1 = list(carry[:_NG])
            cm2 = list(carry[_NG:])
            for l in range(_NG):
                x = buf[rr, pl.ds(16 * l, _L)]
                cm2[l] = jnp.maximum(cm2[l], jnp.minimum(cm1[l], x))
                cm1[l] = jnp.maximum(cm1[l], x)
            return tuple(cm1) + tuple(cm2)

        carry = lax.fori_loop(0, _CT, body, tuple(m1) + tuple(m2))
        m1 = list(carry[:_NG])
        m2 = list(carry[_NG:])

    acc = zeros
    for l in range(_NG):
        l1 = jnp.maximum(m1[l] - tv[l] + _DELTA, 0.0)
        l2 = jnp.maximum(m2[l] - tv[l] + _DELTA, 0.0)
        acc = acc + (l1 + l2)
    acc = acc * (2.0 * _R)

    out_v[...] = acc
    pltpu.sync_copy(out_v, out_hbm.at[pl.ds(wid * _L, _L)])


def kernel(dot_p, target):
    partials = _npt_loss_sc(dot_p.T, target.astype(jnp.int32))
    return jnp.sum(partials) / _B
